# baseline (speedup 1.0000x reference)
import functools

import jax
import jax.numpy as jnp
from jax import lax
from jax.experimental import pallas as pl
from jax.experimental.pallas import tpu as pltpu
from jax.experimental.pallas import tpu_sc as plsc

_SEQ = 8192
_DIM = 64
_NC = 2
_NS = 16
_NW = _NC * _NS
_BPW = _SEQ // _NW


def _body(x_hbm, tok_hbm, pos_hbm, out_hbm, idx_v, pidx_v, acc_v, sem):
    wid = lax.axis_index("s") * _NC + lax.axis_index("c")
    base = wid * _BPW
    pltpu.sync_copy(x_hbm.at[pl.ds(base, _BPW)], idx_v)
    top = _SEQ - 1 - base
    for t in range(_BPW // 16):
        pidx_v[pl.ds(t * 16, 16)] = (top - t * 16) - lax.iota(jnp.int32, 16)
    pltpu.async_copy(pos_hbm.at[pidx_v], acc_v, sem).wait()
    pltpu.async_copy(tok_hbm.at[idx_v], acc_v, sem, add=True).wait()
    pltpu.sync_copy(acc_v, out_hbm.at[pl.ds(base, _BPW)])


@jax.jit
def kernel(x, token_table, pos_table):
    run = pl.kernel(
        _body,
        out_type=jax.ShapeDtypeStruct((_SEQ, _DIM), jnp.float32),
        mesh=plsc.VectorSubcoreMesh(
            core_axis_name="c", subcore_axis_name="s",
            num_cores=_NC, num_subcores=_NS,
        ),
        scratch_types=[
            pltpu.VMEM((_BPW,), jnp.int32),
            pltpu.VMEM((_BPW,), jnp.int32),
            pltpu.VMEM((_BPW, _DIM), jnp.float32),
            pltpu.SemaphoreType.DMA,
        ],
        compiler_params=pltpu.CompilerParams(use_tc_tiling_on_sc=False),
    )
    return run(x.astype(jnp.int32), token_table, pos_table)
